# Initial kernel scaffold; baseline (speedup 1.0000x reference)
import jax
import jax.numpy as jnp
from jax.experimental import pallas as pl


def kernel(dense_inputs, sparse_inputs, tables, ctr_W0, ctr_b0, ctr_W1, ctr_b1, ctr_W2, ctr_b2, ctr_W3, ctr_b3, cvr_W0, cvr_b0, cvr_W1, cvr_b1, cvr_W2, cvr_b2, cvr_W3, cvr_b3):
    raise NotImplementedError("write your pallas kernel here")


# speedup vs baseline: 7.7426x; 7.7426x over previous
import functools

import jax
import jax.numpy as jnp
from jax import lax
from jax.experimental import pallas as pl
from jax.experimental.pallas import tpu as pltpu
from jax.experimental.pallas import tpu_sc as plsc

B = 16384
D_DENSE = 13
F = 26
V = 100000
E = 16

NC = 2
NS = 16
NW = NC * NS
ROWS = B * F
R_PER_W = ROWS // NW
CHUNK = 1664
NCH = R_PER_W // CHUNK

@functools.cache
def _make_sc_gather():
    sc_mesh = plsc.VectorSubcoreMesh(core_axis_name="c", subcore_axis_name="s")

    @functools.partial(
        pl.kernel,
        mesh=sc_mesh,
        compiler_params=pltpu.CompilerParams(use_tc_tiling_on_sc=False),
        out_type=jax.ShapeDtypeStruct((ROWS, E), jnp.float32),
        scratch_types=[
            pltpu.VMEM((NCH, CHUNK), jnp.int32),
            pltpu.VMEM((CHUNK, E), jnp.float32),
            pltpu.VMEM((CHUNK, E), jnp.float32),
            pltpu.SemaphoreType.DMA,
            pltpu.SemaphoreType.DMA,
        ],
    )
    def sc_gather(table_hbm, idx_hbm, out_hbm, idx_v, buf0, buf1, sem0, sem1):
        wid = lax.axis_index("s") * NC + lax.axis_index("c")
        base = wid * R_PER_W
        pltpu.sync_copy(idx_hbm.at[wid], idx_v)
        bufs = (buf0, buf1)
        sems = (sem0, sem1)
        copies = [None, None]
        copies[0] = pltpu.async_copy(table_hbm.at[idx_v.at[0]], bufs[0], sems[0])
        for c in range(NCH):
            if c + 1 < NCH:
                copies[(c + 1) % 2] = pltpu.async_copy(
                    table_hbm.at[idx_v.at[c + 1]], bufs[(c + 1) % 2], sems[(c + 1) % 2])
            copies[c % 2].wait()
            pltpu.sync_copy(bufs[c % 2], out_hbm.at[pl.ds(base + c * CHUNK, CHUNK)])

    return sc_gather


BB = 2048
_GRID = B // BB


def _mlp_body(dense_ref, emb_ref,
              cw0d, cw0e, cb0, cw1, cb1, cw2, cb2, cw3, cb3,
              vw0d, vw0e, vb0, vw1, vb1, vw2, vb2, vw3, vb3,
              ctr_out, ctcvr_out):
    x_d = dense_ref[...]
    x_e = emb_ref[...]

    def tower(w0d, w0e, b0, w1, b1, w2, b2, w3, b3):
        h = jnp.dot(x_d, w0d[...], preferred_element_type=jnp.float32)
        h = h + jnp.dot(x_e, w0e[...], preferred_element_type=jnp.float32)
        h = jnp.maximum(h + b0[...], 0.0)
        h = jnp.maximum(jnp.dot(h, w1[...], preferred_element_type=jnp.float32) + b1[...], 0.0)
        h = jnp.maximum(jnp.dot(h, w2[...], preferred_element_type=jnp.float32) + b2[...], 0.0)
        return jnp.dot(h, w3[...], preferred_element_type=jnp.float32) + b3[...]

    ctr = jax.nn.sigmoid(tower(cw0d, cw0e, cb0, cw1, cb1, cw2, cb2, cw3, cb3))
    cvr = jax.nn.sigmoid(tower(vw0d, vw0e, vb0, vw1, vb1, vw2, vb2, vw3, vb3))
    ctr_out[...] = ctr
    ctcvr_out[...] = ctr * cvr


def _mlp(dense_inputs, emb, ws):
    full = lambda shape: pl.BlockSpec(shape, lambda i: (0, 0))
    in_specs = [
        pl.BlockSpec((BB, D_DENSE), lambda i: (i, 0)),
        pl.BlockSpec((BB, F * E), lambda i: (i, 0)),
    ]
    for _ in range(2):
        in_specs += [
            full((D_DENSE, 256)), full((F * E, 256)), full((1, 256)),
            full((256, 128)), full((1, 128)),
            full((128, 64)), full((1, 64)),
            full((64, 1)), full((1, 1)),
        ]
    return pl.pallas_call(
        _mlp_body,
        grid=(_GRID,),
        in_specs=in_specs,
        out_specs=[pl.BlockSpec((BB, 1), lambda i: (i, 0)),
                   pl.BlockSpec((BB, 1), lambda i: (i, 0))],
        out_shape=[jax.ShapeDtypeStruct((B, 1), jnp.float32),
                   jax.ShapeDtypeStruct((B, 1), jnp.float32)],
    )(dense_inputs, emb, *ws)


def kernel(dense_inputs, sparse_inputs, tables,
           ctr_W0, ctr_b0, ctr_W1, ctr_b1, ctr_W2, ctr_b2, ctr_W3, ctr_b3,
           cvr_W0, cvr_b0, cvr_W1, cvr_b1, cvr_W2, cvr_b2, cvr_W3, cvr_b3):
    table_flat = tables.reshape(F * V, E)
    flat_idx = (sparse_inputs + (jnp.arange(F, dtype=jnp.int32) * V)[None, :]
                ).reshape(NW, NCH, CHUNK)
    emb = _make_sc_gather()(table_flat, flat_idx).reshape(B, F * E)

    def prep(W0, b0, W1, b1, W2, b2, W3, b3):
        return (W0[:D_DENSE], W0[D_DENSE:], b0.reshape(1, -1),
                W1, b1.reshape(1, -1), W2, b2.reshape(1, -1),
                W3, b3.reshape(1, -1))

    ws = (prep(ctr_W0, ctr_b0, ctr_W1, ctr_b1, ctr_W2, ctr_b2, ctr_W3, ctr_b3)
          + prep(cvr_W0, cvr_b0, cvr_W1, cvr_b1, cvr_W2, cvr_b2, cvr_W3, cvr_b3))
    ctr, ctcvr = _mlp(dense_inputs, emb, ws)
    return (ctr, ctcvr)


# speedup vs baseline: 67.6302x; 8.7348x over previous
import functools

import jax
import jax.numpy as jnp
from jax import lax
from jax.experimental import pallas as pl
from jax.experimental.pallas import tpu as pltpu
from jax.experimental.pallas import tpu_sc as plsc

B = 16384
D_DENSE = 13
F = 26
V = 100000
E = 16

NC = 2
NS = 16
NW = NC * NS
R = F * E
R_PER_W = R // NW
IC = 2048
NIC = B // IC


@functools.cache
def _make_sc_gather():
    sc_mesh = plsc.VectorSubcoreMesh(core_axis_name="c", subcore_axis_name="s")

    @functools.partial(
        pl.kernel,
        mesh=sc_mesh,
        compiler_params=pltpu.CompilerParams(use_tc_tiling_on_sc=True,
                                             needs_layout_passes=False),
        out_type=jax.ShapeDtypeStruct((R, B), jnp.float32),
        scratch_types=[
            pltpu.VMEM((V,), jnp.float32),
            pltpu.VMEM((B,), jnp.int32),
            pltpu.VMEM((IC,), jnp.float32),
            pltpu.VMEM((IC,), jnp.float32),
            pltpu.SemaphoreType.DMA,
            pltpu.SemaphoreType.DMA,
            pltpu.SemaphoreType.DMA,
        ],
    )
    def sc_gather(table_hbm, idx_hbm, out_hbm, row_v, idx_v, out_v0, out_v1,
                  row_sem, os0, os1):
        wid = lax.axis_index("s") * NC + lax.axis_index("c")
        r0 = wid * R_PER_W
        outv = (out_v0, out_v1)
        osem = (os0, os1)

        pltpu.async_copy(table_hbm.at[r0], row_v, row_sem)
        pltpu.sync_copy(idx_hbm.at[r0 // E], idx_v)

        def row_body(j, f_prev):
            r = r0 + j
            f = r // E
            pltpu.make_async_copy(table_hbm.at[r], row_v, row_sem).wait()

            @pl.when(f != f_prev)
            def _():
                pltpu.sync_copy(idx_hbm.at[f], idx_v)

            for c in range(NIC):
                p = c % 2

                @pl.when((j > 0) | (c >= 2))
                def _():
                    pltpu.make_async_copy(
                        outv[p], out_hbm.at[r, pl.ds(c * IC, IC)], osem[p]).wait()

                @plsc.parallel_loop(0, IC, 16, unroll=8)
                def gat(i, c=c, p=p):
                    iv = idx_v[pl.ds(c * IC + i, 16)]
                    outv[p][pl.ds(i, 16)] = plsc.load_gather(row_v, [iv])
                if c == NIC - 1:
                    @pl.when(j + 1 < R_PER_W)
                    def _():
                        pltpu.async_copy(table_hbm.at[r + 1], row_v, row_sem)
                pltpu.async_copy(outv[p], out_hbm.at[r, pl.ds(c * IC, IC)], osem[p])
            return f

        f_last = lax.fori_loop(0, R_PER_W, row_body, r0 // E)
        pltpu.make_async_copy(outv[0], out_hbm.at[r0, pl.ds(0, IC)], osem[0]).wait()
        pltpu.make_async_copy(outv[1], out_hbm.at[r0, pl.ds(IC, IC)], osem[1]).wait()
        del f_last

    return sc_gather


BB = 2048
_GRID = B // BB


def _mlp_body(dense_ref, emb_ref,
              cw0d, cw0e, cb0, cw1, cb1, cw2, cb2, cw3, cb3,
              vw0d, vw0e, vb0, vw1, vb1, vw2, vb2, vw3, vb3,
              ctr_out, ctcvr_out):
    x_d = dense_ref[...]
    x_e = emb_ref[...]

    def tower(w0d, w0e, b0, w1, b1, w2, b2, w3, b3):
        h = jnp.dot(w0e[...], x_e, preferred_element_type=jnp.float32)
        h = h + jnp.dot(w0d[...], x_d, preferred_element_type=jnp.float32)
        h = jnp.maximum(h + b0[...], 0.0)
        h = jnp.maximum(jnp.dot(w1[...], h, preferred_element_type=jnp.float32) + b1[...], 0.0)
        h = jnp.maximum(jnp.dot(w2[...], h, preferred_element_type=jnp.float32) + b2[...], 0.0)
        return jnp.dot(w3[...], h, preferred_element_type=jnp.float32) + b3[...]

    ctr = jax.nn.sigmoid(tower(cw0d, cw0e, cb0, cw1, cb1, cw2, cb2, cw3, cb3))
    cvr = jax.nn.sigmoid(tower(vw0d, vw0e, vb0, vw1, vb1, vw2, vb2, vw3, vb3))
    ctr_out[...] = ctr
    ctcvr_out[...] = ctr * cvr


def _mlp(dense_t, emb_t, ws):
    full = lambda shape: pl.BlockSpec(shape, lambda i: (0, 0))
    in_specs = [
        pl.BlockSpec((D_DENSE, BB), lambda i: (0, i)),
        pl.BlockSpec((R, BB), lambda i: (0, i)),
    ]
    for _ in range(2):
        in_specs += [
            full((256, D_DENSE)), full((256, R)), full((256, 1)),
            full((128, 256)), full((128, 1)),
            full((64, 128)), full((64, 1)),
            full((1, 64)), full((1, 1)),
        ]
    return pl.pallas_call(
        _mlp_body,
        grid=(_GRID,),
        in_specs=in_specs,
        out_specs=[pl.BlockSpec((1, BB), lambda i: (0, i)),
                   pl.BlockSpec((1, BB), lambda i: (0, i))],
        out_shape=[jax.ShapeDtypeStruct((1, B), jnp.float32),
                   jax.ShapeDtypeStruct((1, B), jnp.float32)],
    )(dense_t, emb_t, *ws)


def kernel(dense_inputs, sparse_inputs, tables,
           ctr_W0, ctr_b0, ctr_W1, ctr_b1, ctr_W2, ctr_b2, ctr_W3, ctr_b3,
           cvr_W0, cvr_b0, cvr_W1, cvr_b1, cvr_W2, cvr_b2, cvr_W3, cvr_b3):
    table_t = tables.transpose(0, 2, 1).reshape(R, V)
    idx_t = sparse_inputs.T
    emb_t = _make_sc_gather()(table_t, idx_t)

    def prep(W0, b0, W1, b1, W2, b2, W3, b3):
        return (W0[:D_DENSE].T, W0[D_DENSE:].T, b0.reshape(-1, 1),
                W1.T, b1.reshape(-1, 1), W2.T, b2.reshape(-1, 1),
                W3.T, b3.reshape(-1, 1))

    ws = (prep(ctr_W0, ctr_b0, ctr_W1, ctr_b1, ctr_W2, ctr_b2, ctr_W3, ctr_b3)
          + prep(cvr_W0, cvr_b0, cvr_W1, cvr_b1, cvr_W2, cvr_b2, cvr_W3, cvr_b3))
    ctr_t, ctcvr_t = _mlp(dense_inputs.T, emb_t, ws)
    return (ctr_t.reshape(B, 1), ctcvr_t.reshape(B, 1))


# speedup vs baseline: 68.3024x; 1.0099x over previous
import functools

import jax
import jax.numpy as jnp
from jax import lax
from jax.experimental import pallas as pl
from jax.experimental.pallas import tpu as pltpu
from jax.experimental.pallas import tpu_sc as plsc

B = 16384
D_DENSE = 13
F = 26
V = 100000
E = 16

NC = 2
NS = 16
NW = NC * NS
R = F * E
R_PER_W = R // NW
IC = 2048
NIC = B // IC


@functools.cache
def _make_sc_gather():
    sc_mesh = plsc.VectorSubcoreMesh(core_axis_name="c", subcore_axis_name="s")

    @functools.partial(
        pl.kernel,
        mesh=sc_mesh,
        compiler_params=pltpu.CompilerParams(use_tc_tiling_on_sc=True,
                                             needs_layout_passes=False),
        out_type=jax.ShapeDtypeStruct((R, B), jnp.float32),
        scratch_types=[
            pltpu.VMEM((V,), jnp.float32),
            pltpu.VMEM((B,), jnp.int32),
            pltpu.VMEM((IC,), jnp.float32),
            pltpu.VMEM((IC,), jnp.float32),
            pltpu.SemaphoreType.DMA,
            pltpu.SemaphoreType.DMA,
            pltpu.SemaphoreType.DMA,
        ],
    )
    def sc_gather(table_hbm, idx_hbm, out_hbm, row_v, idx_v, out_v0, out_v1,
                  row_sem, os0, os1):
        wid = lax.axis_index("s") * NC + lax.axis_index("c")
        r0 = wid * R_PER_W
        outv = (out_v0, out_v1)
        osem = (os0, os1)

        pltpu.async_copy(table_hbm.at[r0], row_v, row_sem)
        pltpu.sync_copy(idx_hbm.at[r0 // E], idx_v)

        def row_body(j, f_prev):
            r = r0 + j
            f = r // E
            pltpu.make_async_copy(table_hbm.at[r], row_v, row_sem).wait()

            @pl.when(f != f_prev)
            def _():
                pltpu.sync_copy(idx_hbm.at[f], idx_v)

            for c in range(NIC):
                p = c % 2

                @pl.when((j > 0) | (c >= 2))
                def _():
                    pltpu.make_async_copy(
                        outv[p], out_hbm.at[r, pl.ds(c * IC, IC)], osem[p]).wait()

                @plsc.parallel_loop(0, IC, 16, unroll=8)
                def gat(i, c=c, p=p):
                    iv = idx_v[pl.ds(c * IC + i, 16)]
                    outv[p][pl.ds(i, 16)] = plsc.load_gather(row_v, [iv])
                if c == NIC - 1:
                    @pl.when(j + 1 < R_PER_W)
                    def _():
                        pltpu.async_copy(table_hbm.at[r + 1], row_v, row_sem)
                pltpu.async_copy(outv[p], out_hbm.at[r, pl.ds(c * IC, IC)], osem[p])
            return f

        f_last = lax.fori_loop(0, R_PER_W, row_body, r0 // E)
        pltpu.make_async_copy(outv[0], out_hbm.at[r0, pl.ds(0, IC)], osem[0]).wait()
        pltpu.make_async_copy(outv[1], out_hbm.at[r0, pl.ds(IC, IC)], osem[1]).wait()
        del f_last

    return sc_gather


BB = 2048
_GRID = B // BB


def _mlp_body(dense_ref, emb_ref,
              cw0d, cw0e, cb0, cw1, cb1, cw2, cb2, cw3, cb3,
              vw0d, vw0e, vb0, vw1, vb1, vw2, vb2, vw3, vb3,
              ctr_out, ctcvr_out):
    x_d = dense_ref[...]
    x_e = emb_ref[...]

    x_e16 = x_e.astype(jnp.bfloat16)

    def tower(w0d, w0e, b0, w1, b1, w2, b2, w3, b3):
        h = jnp.dot(w0e[...], x_e16, preferred_element_type=jnp.float32)
        h = h + jnp.dot(w0d[...], x_d, preferred_element_type=jnp.float32)
        h = jnp.maximum(h + b0[...], 0.0)
        h = jnp.maximum(jnp.dot(w1[...], h, preferred_element_type=jnp.float32) + b1[...], 0.0)
        h = jnp.maximum(jnp.dot(w2[...], h, preferred_element_type=jnp.float32) + b2[...], 0.0)
        return jnp.dot(w3[...], h, preferred_element_type=jnp.float32) + b3[...]

    ctr = jax.nn.sigmoid(tower(cw0d, cw0e, cb0, cw1, cb1, cw2, cb2, cw3, cb3))
    cvr = jax.nn.sigmoid(tower(vw0d, vw0e, vb0, vw1, vb1, vw2, vb2, vw3, vb3))
    ctr_out[...] = ctr
    ctcvr_out[...] = ctr * cvr


def _mlp(dense_t, emb_t, ws):
    full = lambda shape: pl.BlockSpec(shape, lambda i: (0, 0))
    in_specs = [
        pl.BlockSpec((D_DENSE, BB), lambda i: (0, i)),
        pl.BlockSpec((R, BB), lambda i: (0, i)),
    ]
    for _ in range(2):
        in_specs += [
            full((256, D_DENSE)), full((256, R)), full((256, 1)),
            full((128, 256)), full((128, 1)),
            full((64, 128)), full((64, 1)),
            full((1, 64)), full((1, 1)),
        ]
    return pl.pallas_call(
        _mlp_body,
        grid=(_GRID,),
        in_specs=in_specs,
        out_specs=[pl.BlockSpec((1, BB), lambda i: (0, i)),
                   pl.BlockSpec((1, BB), lambda i: (0, i))],
        out_shape=[jax.ShapeDtypeStruct((1, B), jnp.float32),
                   jax.ShapeDtypeStruct((1, B), jnp.float32)],
    )(dense_t, emb_t, *ws)


def kernel(dense_inputs, sparse_inputs, tables,
           ctr_W0, ctr_b0, ctr_W1, ctr_b1, ctr_W2, ctr_b2, ctr_W3, ctr_b3,
           cvr_W0, cvr_b0, cvr_W1, cvr_b1, cvr_W2, cvr_b2, cvr_W3, cvr_b3):
    table_t = tables.transpose(0, 2, 1).reshape(R, V)
    idx_t = sparse_inputs.T
    emb_t = _make_sc_gather()(table_t, idx_t)

    def prep(W0, b0, W1, b1, W2, b2, W3, b3):
        return (W0[:D_DENSE].T, W0[D_DENSE:].T.astype(jnp.bfloat16),
                b0.reshape(-1, 1),
                W1.T, b1.reshape(-1, 1), W2.T, b2.reshape(-1, 1),
                W3.T, b3.reshape(-1, 1))

    ws = (prep(ctr_W0, ctr_b0, ctr_W1, ctr_b1, ctr_W2, ctr_b2, ctr_W3, ctr_b3)
          + prep(cvr_W0, cvr_b0, cvr_W1, cvr_b1, cvr_W2, cvr_b2, cvr_W3, cvr_b3))
    ctr_t, ctcvr_t = _mlp(dense_inputs.T, emb_t, ws)
    return (ctr_t.reshape(B, 1), ctcvr_t.reshape(B, 1))


# speedup vs baseline: 69.5902x; 1.0189x over previous
import functools

import jax
import jax.numpy as jnp
from jax import lax
from jax.experimental import pallas as pl
from jax.experimental.pallas import tpu as pltpu
from jax.experimental.pallas import tpu_sc as plsc

B = 16384
D_DENSE = 13
F = 26
V = 100000
E = 16

NC = 2
NS = 16
NW = NC * NS
R = F * E
R_PER_W = R // NW
IC = 4096
NIC = B // IC


@functools.cache
def _make_sc_gather():
    sc_mesh = plsc.VectorSubcoreMesh(core_axis_name="c", subcore_axis_name="s")

    @functools.partial(
        pl.kernel,
        mesh=sc_mesh,
        compiler_params=pltpu.CompilerParams(use_tc_tiling_on_sc=True,
                                             needs_layout_passes=False),
        out_type=jax.ShapeDtypeStruct((R, B), jnp.float32),
        scratch_types=[
            pltpu.VMEM((V,), jnp.float32),
            pltpu.VMEM((B,), jnp.int32),
            pltpu.VMEM((IC,), jnp.float32),
            pltpu.VMEM((IC,), jnp.float32),
            pltpu.SemaphoreType.DMA,
            pltpu.SemaphoreType.DMA,
            pltpu.SemaphoreType.DMA,
        ],
    )
    def sc_gather(table_hbm, idx_hbm, out_hbm, row_v, idx_v, out_v0, out_v1,
                  row_sem, os0, os1):
        wid = lax.axis_index("s") * NC + lax.axis_index("c")
        r0 = wid * R_PER_W
        outv = (out_v0, out_v1)
        osem = (os0, os1)

        pltpu.async_copy(table_hbm.at[r0], row_v, row_sem)
        pltpu.sync_copy(idx_hbm.at[r0 // E], idx_v)

        def row_body(j, f_prev):
            r = r0 + j
            f = r // E
            pltpu.make_async_copy(table_hbm.at[r], row_v, row_sem).wait()

            @pl.when(f != f_prev)
            def _():
                pltpu.sync_copy(idx_hbm.at[f], idx_v)

            for c in range(NIC):
                p = c % 2

                @pl.when((j > 0) | (c >= 2))
                def _():
                    pltpu.make_async_copy(
                        outv[p], out_hbm.at[r, pl.ds(c * IC, IC)], osem[p]).wait()

                @plsc.parallel_loop(0, IC, 16, unroll=8)
                def gat(i, c=c, p=p):
                    iv = idx_v[pl.ds(c * IC + i, 16)]
                    outv[p][pl.ds(i, 16)] = plsc.load_gather(row_v, [iv])
                if c == NIC - 1:
                    @pl.when(j + 1 < R_PER_W)
                    def _():
                        pltpu.async_copy(table_hbm.at[r + 1], row_v, row_sem)
                pltpu.async_copy(outv[p], out_hbm.at[r, pl.ds(c * IC, IC)], osem[p])
            return f

        f_last = lax.fori_loop(0, R_PER_W, row_body, r0 // E)
        pltpu.make_async_copy(outv[0], out_hbm.at[r0, pl.ds(0, IC)], osem[0]).wait()
        pltpu.make_async_copy(outv[1], out_hbm.at[r0, pl.ds(IC, IC)], osem[1]).wait()
        del f_last

    return sc_gather


BB = 4096
_GRID = B // BB


def _mlp_body(dense_ref, emb_ref,
              cw0d, cw0e, cb0, cw1, cb1, cw2, cb2, cw3, cb3,
              vw0d, vw0e, vb0, vw1, vb1, vw2, vb2, vw3, vb3,
              ctr_out, ctcvr_out):
    x_d = dense_ref[...]
    x_e = emb_ref[...]

    x_e16 = x_e.astype(jnp.bfloat16)

    def tower(w0d, w0e, b0, w1, b1, w2, b2, w3, b3):
        h = jnp.dot(w0e[...], x_e16, preferred_element_type=jnp.float32)
        h = h + jnp.dot(w0d[...], x_d, preferred_element_type=jnp.float32)
        h = jnp.maximum(h + b0[...], 0.0)
        h = jnp.maximum(jnp.dot(w1[...], h, preferred_element_type=jnp.float32) + b1[...], 0.0)
        h = jnp.maximum(jnp.dot(w2[...], h, preferred_element_type=jnp.float32) + b2[...], 0.0)
        return jnp.dot(w3[...], h, preferred_element_type=jnp.float32) + b3[...]

    ctr = jax.nn.sigmoid(tower(cw0d, cw0e, cb0, cw1, cb1, cw2, cb2, cw3, cb3))
    cvr = jax.nn.sigmoid(tower(vw0d, vw0e, vb0, vw1, vb1, vw2, vb2, vw3, vb3))
    ctr_out[...] = ctr
    ctcvr_out[...] = ctr * cvr


def _mlp(dense_t, emb_t, ws):
    full = lambda shape: pl.BlockSpec(shape, lambda i: (0, 0))
    in_specs = [
        pl.BlockSpec((D_DENSE, BB), lambda i: (0, i)),
        pl.BlockSpec((R, BB), lambda i: (0, i)),
    ]
    for _ in range(2):
        in_specs += [
            full((256, D_DENSE)), full((256, R)), full((256, 1)),
            full((128, 256)), full((128, 1)),
            full((64, 128)), full((64, 1)),
            full((1, 64)), full((1, 1)),
        ]
    return pl.pallas_call(
        _mlp_body,
        grid=(_GRID,),
        in_specs=in_specs,
        out_specs=[pl.BlockSpec((1, BB), lambda i: (0, i)),
                   pl.BlockSpec((1, BB), lambda i: (0, i))],
        out_shape=[jax.ShapeDtypeStruct((1, B), jnp.float32),
                   jax.ShapeDtypeStruct((1, B), jnp.float32)],
    )(dense_t, emb_t, *ws)


def kernel(dense_inputs, sparse_inputs, tables,
           ctr_W0, ctr_b0, ctr_W1, ctr_b1, ctr_W2, ctr_b2, ctr_W3, ctr_b3,
           cvr_W0, cvr_b0, cvr_W1, cvr_b1, cvr_W2, cvr_b2, cvr_W3, cvr_b3):
    table_t = tables.transpose(0, 2, 1).reshape(R, V)
    idx_t = sparse_inputs.T
    emb_t = _make_sc_gather()(table_t, idx_t)

    def prep(W0, b0, W1, b1, W2, b2, W3, b3):
        return (W0[:D_DENSE].T, W0[D_DENSE:].T.astype(jnp.bfloat16),
                b0.reshape(-1, 1),
                W1.T, b1.reshape(-1, 1), W2.T, b2.reshape(-1, 1),
                W3.T, b3.reshape(-1, 1))

    ws = (prep(ctr_W0, ctr_b0, ctr_W1, ctr_b1, ctr_W2, ctr_b2, ctr_W3, ctr_b3)
          + prep(cvr_W0, cvr_b0, cvr_W1, cvr_b1, cvr_W2, cvr_b2, cvr_W3, cvr_b3))
    ctr_t, ctcvr_t = _mlp(dense_inputs.T, emb_t, ws)
    return (ctr_t.reshape(B, 1), ctcvr_t.reshape(B, 1))
